# baseline (speedup 1.0000x reference)
import jax
import jax.numpy as jnp
from jax import lax
from jax.experimental import pallas as pl
from jax.experimental.pallas import tpu as pltpu
from jax.experimental.pallas import tpu_sc as plsc

D = 64
NC = 2
NS = 16
NW = NC * NS
B = 4096
S = 200
LANES = 128
V = 1000000
VROWS = V // 2
FULL_PANELS = 7812
TAIL_V = V - FULL_PANELS * LANES


def _fmt_body(tabT_hbm, tail_hbm, out_hbm, buf, ob, sem_g, sem_w):
    w = lax.axis_index("s") * NC + lax.axis_index("c")
    n_my = jnp.where(w < FULL_PANELS - (FULL_PANELS // NW) * NW,
                     FULL_PANELS // NW + 1, FULL_PANELS // NW)

    def fire(k, q):
        pid = w + NW * k
        pltpu.async_copy(
            tabT_hbm.at[:, pl.ds(pid * LANES, LANES)], buf.at[q], sem_g
        )

    def panel(k, bufp, obp):
        pid = w + NW * k
        for j in range(64):
            for cg in range(8):
                dv = lax.iota(jnp.int32, 16) + (cg % 4) * 16
                vv = jnp.full((16,), 2 * j + cg // 4, jnp.int32)
                g = plsc.load_gather(bufp, [dv, vv])
                obp[j, pl.ds(cg * 16, 16)] = g
        pltpu.async_copy(obp, out_hbm.at[pl.ds(pid * 64, 64)], sem_w)

    fire(0, 0)

    def step(k, carry):
        p = lax.rem(k, 2)

        @pl.when(k + 1 < n_my)
        def _():
            fire(k + 1, 1 - p)

        pltpu.make_async_copy(
            tabT_hbm.at[:, pl.ds(0, LANES)], buf.at[p], sem_g
        ).wait()

        @pl.when(k >= 2)
        def _():
            pltpu.make_async_copy(ob.at[0], out_hbm.at[pl.ds(0, 64)], sem_w).wait()

        @pl.when(p == 0)
        def _():
            panel(k, buf.at[0], ob.at[0])

        @pl.when(p == 1)
        def _():
            panel(k, buf.at[1], ob.at[1])

        return carry

    lax.fori_loop(0, n_my, step, 0)
    pltpu.make_async_copy(ob.at[0], out_hbm.at[pl.ds(0, 64)], sem_w).wait()
    pltpu.make_async_copy(ob.at[0], out_hbm.at[pl.ds(0, 64)], sem_w).wait()

    @pl.when(w == 0)
    def _():
        pltpu.sync_copy(tail_hbm, ob.at[0, pl.ds(0, TAIL_V // 2)])
        pltpu.sync_copy(
            ob.at[0, pl.ds(0, TAIL_V // 2)],
            out_hbm.at[pl.ds(FULL_PANELS * 64, TAIL_V // 2)],
        )


def _emb_body(xT_hbm, tab_hbm, out_hbm, idx_v, idx2_v, buf, tb, sem_g, sem_w):
    w = lax.axis_index("s") * NC + lax.axis_index("c")
    base = w * LANES
    pltpu.sync_copy(xT_hbm.at[:, pl.ds(base, LANES)], idx_v)

    def prep(g, carry):
        row = g // 8
        col = (g % 8) * 16
        v = idx_v[row, pl.ds(col, 16)]
        idx2_v[row, pl.ds(col, 16)] = lax.shift_right_logical(v, 1)
        return carry

    lax.fori_loop(0, S * 8, prep, 0)

    def fire(s, q):
        pltpu.async_copy(tab_hbm.at[idx2_v.at[s]], buf.at[q], sem_g)

    fire(0, 0)
    flat_base = lax.iota(jnp.int32, 16) * 128
    zero16 = jnp.zeros((16,), jnp.int32)

    def transpose_block(s, bufp, tbp):
        for rg in range(8):
            pc = (idx_v[s, pl.ds(rg * 16, 16)] & 1) * D
            av = flat_base + (rg * 16 * 128) + pc
            for dg in range(8):
                gs = [
                    plsc.load_gather(bufp, [zero16, av + (dg * 8 + j)])
                    for j in range(8)
                ]
                for j in range(8):
                    tbp[dg * 8 + j, pl.ds(rg * 16, 16)] = gs[j]
        pltpu.async_copy(tbp, out_hbm.at[s, :, pl.ds(base, LANES)], sem_w)

    def step(s, carry):
        p = lax.rem(s, 2)

        @pl.when(s + 1 < S)
        def _():
            fire(s + 1, 1 - p)

        pltpu.make_async_copy(tab_hbm.at[pl.ds(0, LANES)], buf.at[p], sem_g).wait()

        @pl.when(s >= 2)
        def _():
            pltpu.make_async_copy(
                tb.at[0], out_hbm.at[0, :, pl.ds(0, LANES)], sem_w
            ).wait()

        @pl.when(p == 0)
        def _():
            transpose_block(s, buf.at[0], tb.at[0])

        @pl.when(p == 1)
        def _():
            transpose_block(s, buf.at[1], tb.at[1])

        return carry

    lax.fori_loop(0, S, step, 0)
    pltpu.make_async_copy(tb.at[0], out_hbm.at[0, :, pl.ds(0, LANES)], sem_w).wait()
    pltpu.make_async_copy(tb.at[0], out_hbm.at[0, :, pl.ds(0, LANES)], sem_w).wait()


def kernel(x, table):
    xT = jnp.transpose(x.astype(jnp.int32), (1, 0))
    tabT = jnp.transpose(table, (1, 0))
    tail = table[V - TAIL_V:].reshape(TAIL_V // 2, 128)
    mesh = plsc.VectorSubcoreMesh(core_axis_name="c", subcore_axis_name="s")
    params = pltpu.CompilerParams(
        use_tc_tiling_on_sc=True,
        needs_layout_passes=False,
        disable_bounds_checks=True,
    )
    tab2 = pl.kernel(
        _fmt_body,
        out_type=jax.ShapeDtypeStruct((VROWS, 128), jnp.float32),
        mesh=mesh,
        scratch_types=[
            pltpu.VMEM((2, D, LANES), jnp.float32),
            pltpu.VMEM((2, D, LANES), jnp.float32),
            pltpu.SemaphoreType.DMA,
            pltpu.SemaphoreType.DMA,
        ],
        compiler_params=params,
    )(tabT, tail)
    out = pl.kernel(
        _emb_body,
        out_type=jax.ShapeDtypeStruct((S, D, B), jnp.float32),
        mesh=mesh,
        scratch_types=[
            pltpu.VMEM((S, LANES), jnp.int32),
            pltpu.VMEM((S, LANES), jnp.int32),
            pltpu.VMEM((2, LANES, 128), jnp.float32),
            pltpu.VMEM((2, D, LANES), jnp.float32),
            pltpu.SemaphoreType.DMA,
            pltpu.SemaphoreType.DMA,
        ],
        compiler_params=params,
    )(xT, tab2)
    return jnp.transpose(out, (2, 0, 1))


# speedup vs baseline: 1.0109x; 1.0109x over previous
import jax
import jax.numpy as jnp
from jax import lax
from jax.experimental import pallas as pl
from jax.experimental.pallas import tpu as pltpu
from jax.experimental.pallas import tpu_sc as plsc

D = 64
NC = 2
NS = 16
NW = NC * NS
B = 4096
S = 200
LANES = 128
V = 1000000
VROWS = V // 2
PANEL = 256
FULL_PANELS = V // PANEL
TAIL_V = V - FULL_PANELS * PANEL


def _fmt_body(tabT_hbm, tail_hbm, out_hbm, buf, ob, sem_g, sem_w):
    w = lax.axis_index("s") * NC + lax.axis_index("c")
    extra = FULL_PANELS - (FULL_PANELS // NW) * NW
    n_my = jnp.where(w < extra, FULL_PANELS // NW + 1, FULL_PANELS // NW)

    def fire(k):
        pid = w + NW * k
        pltpu.async_copy(
            tabT_hbm.at[:, pl.ds(pid * PANEL, PANEL)],
            buf.at[lax.rem(k, 4)],
            sem_g,
        )

    for k0 in range(3):
        @pl.when(k0 < n_my)
        def _():
            fire(k0)

    iota16 = lax.iota(jnp.int32, 16)
    dvs = [(cg % 4) * 16 + iota16 for cg in range(8)]

    def step(k, carry):
        q = lax.rem(k, 4)
        t = lax.rem(k, 2)
        pid = w + NW * k

        @pl.when(k + 3 < n_my)
        def _():
            fire(k + 3)

        pltpu.make_async_copy(
            tabT_hbm.at[:, pl.ds(0, PANEL)], buf.at[q], sem_g
        ).wait()

        @pl.when(k >= 2)
        def _():
            pltpu.make_async_copy(ob.at[0], out_hbm.at[pl.ds(0, 128)], sem_w).wait()

        qv = jnp.full((16,), q, jnp.int32)

        def jgroup(jg, c2):
            jj = jg * 32
            for jl in range(16):
                for cg in range(8):
                    vv = jnp.full((16,), 2 * jl + cg // 4, jnp.int32) + jj
                    g = plsc.load_gather(buf, [qv, dvs[cg], vv])
                    ob[t, jg * 16 + jl, pl.ds(cg * 16, 16)] = g
            return c2

        lax.fori_loop(0, 8, jgroup, 0)
        pltpu.async_copy(ob.at[t], out_hbm.at[pl.ds(pid * 128, 128)], sem_w)
        return carry

    lax.fori_loop(0, n_my, step, 0)
    pltpu.make_async_copy(ob.at[0], out_hbm.at[pl.ds(0, 128)], sem_w).wait()
    pltpu.make_async_copy(ob.at[0], out_hbm.at[pl.ds(0, 128)], sem_w).wait()

    @pl.when(w == 0)
    def _():
        pltpu.sync_copy(tail_hbm, ob.at[0, pl.ds(0, TAIL_V // 2)])
        pltpu.sync_copy(
            ob.at[0, pl.ds(0, TAIL_V // 2)],
            out_hbm.at[pl.ds(FULL_PANELS * 128, TAIL_V // 2)],
        )


def _emb_body(xT_hbm, tab_hbm, out_hbm, idx_v, idx2_v, buf, tb, sem_g, sem_w):
    w = lax.axis_index("s") * NC + lax.axis_index("c")
    base = w * LANES
    pltpu.sync_copy(xT_hbm.at[:, pl.ds(base, LANES)], idx_v)

    def prep(g, carry):
        row = g // 8
        col = (g % 8) * 16
        v = idx_v[row, pl.ds(col, 16)]
        idx2_v[row, pl.ds(col, 16)] = lax.shift_right_logical(v, 1)
        return carry

    lax.fori_loop(0, S * 8, prep, 0)

    def fire(s):
        q = lax.rem(s, 3)
        pltpu.async_copy(tab_hbm.at[idx2_v.at[s]], buf.at[q], sem_g)

    for s0 in range(2):
        fire(s0)

    rbase = lax.iota(jnp.int32, 16)

    def step(s, carry):
        q = lax.rem(s, 3)
        t = lax.rem(s, 2)

        @pl.when(s + 2 < S)
        def _():
            fire(s + 2)

        pltpu.make_async_copy(
            tab_hbm.at[pl.ds(0, LANES)], buf.at[0], sem_g
        ).wait()

        @pl.when(s >= 2)
        def _():
            pltpu.make_async_copy(
                tb.at[0], out_hbm.at[0, :, pl.ds(0, LANES)], sem_w
            ).wait()

        qv = jnp.full((16,), q, jnp.int32)
        for rg in range(8):
            pc = (idx_v[s, pl.ds(rg * 16, 16)] & 1) * D
            rv = rbase + rg * 16
            for dg in range(8):
                gs = [
                    plsc.load_gather(buf, [qv, rv, pc + (dg * 8 + j)])
                    for j in range(8)
                ]
                for j in range(8):
                    tb[t, dg * 8 + j, pl.ds(rg * 16, 16)] = gs[j]

        pltpu.async_copy(tb.at[t], out_hbm.at[s, :, pl.ds(base, LANES)], sem_w)
        return carry

    lax.fori_loop(0, S, step, 0)
    pltpu.make_async_copy(tb.at[0], out_hbm.at[0, :, pl.ds(0, LANES)], sem_w).wait()
    pltpu.make_async_copy(tb.at[0], out_hbm.at[0, :, pl.ds(0, LANES)], sem_w).wait()


def kernel(x, table):
    xT = jnp.transpose(x.astype(jnp.int32), (1, 0))
    tabT = jnp.transpose(table, (1, 0))
    tail = table[V - TAIL_V:].reshape(TAIL_V // 2, 128)
    mesh = plsc.VectorSubcoreMesh(core_axis_name="c", subcore_axis_name="s")
    params = pltpu.CompilerParams(
        use_tc_tiling_on_sc=True,
        needs_layout_passes=False,
        disable_bounds_checks=True,
    )
    tab2 = pl.kernel(
        _fmt_body,
        out_type=jax.ShapeDtypeStruct((VROWS, 128), jnp.float32),
        mesh=mesh,
        scratch_types=[
            pltpu.VMEM((4, D, PANEL), jnp.float32),
            pltpu.VMEM((2, 128, 128), jnp.float32),
            pltpu.SemaphoreType.DMA,
            pltpu.SemaphoreType.DMA,
        ],
        compiler_params=params,
    )(tabT, tail)
    out = pl.kernel(
        _emb_body,
        out_type=jax.ShapeDtypeStruct((S, D, B), jnp.float32),
        mesh=mesh,
        scratch_types=[
            pltpu.VMEM((S, LANES), jnp.int32),
            pltpu.VMEM((S, LANES), jnp.int32),
            pltpu.VMEM((3, LANES, 128), jnp.float32),
            pltpu.VMEM((2, D, LANES), jnp.float32),
            pltpu.SemaphoreType.DMA,
            pltpu.SemaphoreType.DMA,
        ],
        compiler_params=params,
    )(xT, tab2)
    return jnp.transpose(out, (2, 0, 1))


# speedup vs baseline: 1.8445x; 1.8247x over previous
import jax
import jax.numpy as jnp
from jax import lax
from jax.experimental import pallas as pl
from jax.experimental.pallas import tpu as pltpu
from jax.experimental.pallas import tpu_sc as plsc

D = 64
NC = 2
NS = 16
NW = NC * NS
GROUP = 128
N_TOTAL = 4096 * 200
PER_W = N_TOTAL // NW
G = PER_W // GROUP
K = 4
CHUNK = K * GROUP
C = G // K


def _emb_body(x_hbm, table_hbm, out_hbm, idx_v, buf, sem_g, sem_s):
    wid = lax.axis_index("s") * NC + lax.axis_index("c")
    pltpu.sync_copy(x_hbm.at[wid], idx_v)

    def fire_chunk(c, p):
        for j in range(K):
            pltpu.async_copy(
                table_hbm.at[idx_v.at[c * K + j]],
                buf.at[p, pl.ds(j * GROUP, GROUP)],
                sem_g,
            )

    def wait_chunk(p):
        pltpu.make_async_copy(
            table_hbm.at[pl.ds(0, CHUNK)], buf.at[p], sem_g
        ).wait()

    fire_chunk(0, 0)

    def chunk(c, carry):
        p = lax.rem(c, 2)
        @pl.when(c >= 1)
        def _():
            pltpu.make_async_copy(buf.at[1 - p], out_hbm.at[wid, 0], sem_s).wait()

        @pl.when(c + 1 < C)
        def _():
            fire_chunk(c + 1, 1 - p)

        wait_chunk(p)
        pltpu.async_copy(buf.at[p], out_hbm.at[wid, c], sem_s)
        return carry

    lax.fori_loop(0, C, chunk, 0)
    pltpu.make_async_copy(buf.at[0], out_hbm.at[wid, 0], sem_s).wait()


def kernel(x, table):
    x3 = x.reshape(NW, G, GROUP).astype(jnp.int32)
    mesh = plsc.VectorSubcoreMesh(core_axis_name="c", subcore_axis_name="s")
    out = pl.kernel(
        _emb_body,
        out_type=jax.ShapeDtypeStruct((NW, C, CHUNK, D), jnp.float32),
        mesh=mesh,
        scratch_types=[
            pltpu.VMEM((G, GROUP), jnp.int32),
            pltpu.VMEM((2, CHUNK, D), jnp.float32),
            pltpu.SemaphoreType.DMA,
            pltpu.SemaphoreType.DMA,
        ],
        compiler_params=pltpu.CompilerParams(use_tc_tiling_on_sc=False),
    )(x3, table)
    return out.reshape(4096, 200, D)


# speedup vs baseline: 1.8445x; 1.0000x over previous
import jax
import jax.numpy as jnp
from jax import lax
from jax.experimental import pallas as pl
from jax.experimental.pallas import tpu as pltpu
from jax.experimental.pallas import tpu_sc as plsc

D = 64
NC = 2
NS = 16
NW = NC * NS
GROUP = 128
N_TOTAL = 4096 * 200
PER_W = N_TOTAL // NW
G = PER_W // GROUP
K = 4
CHUNK = K * GROUP
C = G // K


def _emb_body(x_hbm, table_hbm, out_hbm, idx_v, buf, sem_g, sem_s):
    wid = lax.axis_index("s") * NC + lax.axis_index("c")
    pltpu.sync_copy(x_hbm.at[wid], idx_v)

    def fire_chunk(c, p):
        for j in range(K):
            pltpu.async_copy(
                table_hbm.at[idx_v.at[c * K + j]],
                buf.at[p, pl.ds(j * GROUP, GROUP)],
                sem_g,
            )

    def wait_chunk(p):
        pltpu.make_async_copy(
            table_hbm.at[pl.ds(0, CHUNK)], buf.at[p], sem_g
        ).wait()

    fire_chunk(0, 0)
    fire_chunk(1, 1)

    def chunk(c, carry):
        p = lax.rem(c, 3)
        @pl.when(c >= 1)
        def _():
            pltpu.make_async_copy(buf.at[0], out_hbm.at[wid, 0], sem_s).wait()

        @pl.when(c + 2 < C)
        def _():
            fire_chunk(c + 2, lax.rem(c + 2, 3))

        wait_chunk(p)
        pltpu.async_copy(buf.at[p], out_hbm.at[wid, c], sem_s)
        return carry

    lax.fori_loop(0, C, chunk, 0)
    pltpu.make_async_copy(buf.at[0], out_hbm.at[wid, 0], sem_s).wait()


def kernel(x, table):
    x3 = x.reshape(NW, G, GROUP).astype(jnp.int32)
    mesh = plsc.VectorSubcoreMesh(core_axis_name="c", subcore_axis_name="s")
    out = pl.kernel(
        _emb_body,
        out_type=jax.ShapeDtypeStruct((NW, C, CHUNK, D), jnp.float32),
        mesh=mesh,
        scratch_types=[
            pltpu.VMEM((G, GROUP), jnp.int32),
            pltpu.VMEM((3, CHUNK, D), jnp.float32),
            pltpu.SemaphoreType.DMA,
            pltpu.SemaphoreType.DMA,
        ],
        compiler_params=pltpu.CompilerParams(use_tc_tiling_on_sc=False),
    )(x3, table)
    return out.reshape(4096, 200, D)
